# baseline (speedup 1.0000x reference)
import functools

import jax
import jax.numpy as jnp
from jax import lax
from jax.experimental import pallas as pl
from jax.experimental.pallas import tpu as pltpu
from jax.experimental.pallas import tpu_sc as plsc

_B = 16384
_KD = 128
_TD = 64
_NC = 2
_NS = 16
_NW = _NC * _NS
_PER_W = _B // _NW
_C = 128
_ROUNDS = _PER_W // _C
_GROUPS = _C // 16
_REG = 1e-05
_HU = 50000
_HI = 25000

_MESH = plsc.VectorSubcoreMesh(
    core_axis_name="c", subcore_axis_name="s",
    num_cores=_NC, num_subcores=_NS)
_PARAMS = pltpu.CompilerParams(needs_layout_passes=False)


def _kgat_body(u_ref, pos_ref, neg_ref, ua_ref, ea_ref,
               o0, o1, o2, o3, o4,
               idx_u, idx_p, idx_n, ru, rp, rn, staging, sem):
  outs = (o0, o1, o2, o3, o4)
  wid = lax.axis_index("s") * _NC + lax.axis_index("c")
  base = wid * _PER_W
  lane = lax.iota(jnp.int32, 16)

  for r in range(_ROUNDS):
    off = base + r * _C
    pltpu.sync_copy(u_ref.at[pl.ds(off, _C)], idx_u)
    pltpu.sync_copy(pos_ref.at[pl.ds(off, _C)], idx_p)
    pltpu.sync_copy(neg_ref.at[pl.ds(off, _C)], idx_n)
    cps = [
        pltpu.async_copy(ua_ref.at[idx_u], ru, sem),
        pltpu.async_copy(ea_ref.at[idx_p], rp, sem),
        pltpu.async_copy(ea_ref.at[idx_n], rn, sem),
    ]
    for cp in cps:
      cp.wait()

    for g in range(_GROUPS):
      rows = lane + (g * 16)
      z = jnp.zeros((16,), jnp.float32)

      def kbody(d, accs, rows=rows):
        aup, aun, auu, app, ann = accs
        col = lax.bitwise_and(lane + d, _KD - 1)
        uv = plsc.load_gather(ru, [rows, col])
        pv = plsc.load_gather(rp, [rows, col])
        nv = plsc.load_gather(rn, [rows, col])
        return (aup + uv * pv, aun + uv * nv,
                auu + uv * uv, app + pv * pv, ann + nv * nv)

      accs = lax.fori_loop(0, _KD, kbody, (z, z, z, z, z), unroll=4)

      o = r * _C + g * 16
      for j, val in enumerate(accs):
        staging[pl.ds(j * _PER_W + o, 16)] = val

  for j in range(5):
    pltpu.sync_copy(staging.at[pl.ds(j * _PER_W, _PER_W)],
                    outs[j].at[pl.ds(base, _PER_W)])


_kgat_stats = functools.partial(
    pl.kernel,
    out_type=tuple(jax.ShapeDtypeStruct((_B,), jnp.float32)
                   for _ in range(5)),
    mesh=_MESH,
    scratch_types=[
        pltpu.VMEM((_C,), jnp.int32),
        pltpu.VMEM((_C,), jnp.int32),
        pltpu.VMEM((_C,), jnp.int32),
        pltpu.VMEM((_C, _KD), jnp.float32),
        pltpu.VMEM((_C, _KD), jnp.float32),
        pltpu.VMEM((_C, _KD), jnp.float32),
        pltpu.VMEM((5 * _PER_W,), jnp.float32),
        pltpu.SemaphoreType.DMA,
    ],
    compiler_params=_PARAMS,
)(_kgat_body)


def _text_body(u_ref, pos_ref, neg_ref, ut_ref, it_ref, wu_ref, wc_ref,
               o0, o1, o2, o3, o4, o5, o6, o7,
               idx_u, idx_p, idx_n, tix_u, tix_p, tix_n,
               tu, tp, tn, wus, wps, wns, staging, sem):
  outs = (o0, o1, o2, o3, o4, o5, o6, o7)
  wid = lax.axis_index("s") * _NC + lax.axis_index("c")
  base = wid * _PER_W
  lane = lax.iota(jnp.int32, 16)

  for r in range(_ROUNDS):
    off = base + r * _C
    pltpu.sync_copy(u_ref.at[pl.ds(off, _C)], idx_u)
    pltpu.sync_copy(pos_ref.at[pl.ds(off, _C)], idx_p)
    pltpu.sync_copy(neg_ref.at[pl.ds(off, _C)], idx_n)
    for g in range(_GROUPS):
      sl = pl.ds(g * 16, 16)
      vu, vp, vn = idx_u[sl], idx_p[sl], idx_n[sl]
      tix_u[sl] = jnp.where(vu < _HU, vu, vu - _HU)
      tix_p[sl] = jnp.where(vp < _HI, vp, vp - _HI)
      tix_n[sl] = jnp.where(vn < _HI, vn, vn - _HI)
    cps = [
        pltpu.async_copy(ut_ref.at[tix_u], tu, sem),
        pltpu.async_copy(it_ref.at[tix_p], tp, sem),
        pltpu.async_copy(it_ref.at[tix_n], tn, sem),
        pltpu.async_copy(wu_ref.at[idx_u], wus, sem),
        pltpu.async_copy(wc_ref.at[idx_p], wps, sem),
        pltpu.async_copy(wc_ref.at[idx_n], wns, sem),
    ]
    for cp in cps:
      cp.wait()

    for g in range(_GROUPS):
      rows = lane + (g * 16)
      z = jnp.zeros((16,), jnp.float32)

      half_u = jnp.where(idx_u[pl.ds(g * 16, 16)] < _HU, 0, _TD)
      half_p = jnp.where(idx_p[pl.ds(g * 16, 16)] < _HI, 0, _TD)
      half_n = jnp.where(idx_n[pl.ds(g * 16, 16)] < _HI, 0, _TD)

      def tbody(d, accs, rows=rows, hu=half_u, hp=half_p, hn=half_n):
        bup, bun, buu, bpp, bnn = accs
        rot = lax.bitwise_and(lane + d, _TD - 1)
        uv = plsc.load_gather(tu, [rows, hu + rot])
        pv = plsc.load_gather(tp, [rows, hp + rot])
        nv = plsc.load_gather(tn, [rows, hn + rot])
        return (bup + uv * pv, bun + uv * nv,
                buu + uv * uv, bpp + pv * pv, bnn + nv * nv)

      accs = lax.fori_loop(0, _TD, tbody, (z, z, z, z, z), unroll=4)

      wuv = wus[pl.ds(g * 16, 16)]
      wpv = wps[pl.ds(g * 16, 16)]
      wnv = wns[pl.ds(g * 16, 16)]

      o = r * _C + g * 16
      for j, val in enumerate(accs + (wuv, wpv, wnv)):
        staging[pl.ds(j * _PER_W + o, 16)] = val

  for j in range(8):
    pltpu.sync_copy(staging.at[pl.ds(j * _PER_W, _PER_W)],
                    outs[j].at[pl.ds(base, _PER_W)])


_text_stats = functools.partial(
    pl.kernel,
    out_type=tuple(jax.ShapeDtypeStruct((_B,), jnp.float32)
                   for _ in range(8)),
    mesh=_MESH,
    scratch_types=[
        pltpu.VMEM((_C,), jnp.int32),
        pltpu.VMEM((_C,), jnp.int32),
        pltpu.VMEM((_C,), jnp.int32),
        pltpu.VMEM((_C,), jnp.int32),
        pltpu.VMEM((_C,), jnp.int32),
        pltpu.VMEM((_C,), jnp.int32),
        pltpu.VMEM((_C, _KD), jnp.float32),
        pltpu.VMEM((_C, _KD), jnp.float32),
        pltpu.VMEM((_C, _KD), jnp.float32),
        pltpu.VMEM((_C,), jnp.float32),
        pltpu.VMEM((_C,), jnp.float32),
        pltpu.VMEM((_C,), jnp.float32),
        pltpu.VMEM((8 * _PER_W,), jnp.float32),
        pltpu.SemaphoreType.DMA,
    ],
    compiler_params=_PARAMS,
)(_text_body)


def _pack_body(a_ref, b_ref, o_ref):
  o_ref[...] = jnp.concatenate([a_ref[...], b_ref[...]], axis=1)


def _pack_halves(t):
  n = t.shape[0]
  nb = n // 2000
  return pl.pallas_call(
      _pack_body,
      grid=(nb,),
      in_specs=[pl.BlockSpec((1000, _TD), lambda i: (i, 0)),
                pl.BlockSpec((1000, _TD), lambda i, nb=nb: (i + nb, 0))],
      out_specs=pl.BlockSpec((1000, 2 * _TD), lambda i: (i, 0)),
      out_shape=jax.ShapeDtypeStruct((n // 2, 2 * _TD), jnp.float32),
  )(t, t)


def _squeeze_body(x_ref, o_ref):
  x = x_ref[...][:, 0]
  o_ref[...] = jnp.pad(x, (0, o_ref.shape[0] - x.shape[0]))


def _squeeze_w(w):
  n = w.shape[0]
  npad = ((n + 127) // 128) * 128
  return pl.pallas_call(
      _squeeze_body,
      out_shape=jax.ShapeDtypeStruct((npad,), jnp.float32),
  )(w)


def _ep_body(up_r, un_r, uu_r, pp_r, nn_r, tup_r, tun_r, tuu_r, tpp_r,
             tnn_r, wu_r, wp_r, wn_r, o_ref):
  up, un, uu, pp, nn = up_r[...], un_r[...], uu_r[...], pp_r[...], nn_r[...]
  tup, tun, tuu, tpp, tnn = (tup_r[...], tun_r[...], tuu_r[...],
                             tpp_r[...], tnn_r[...])
  wu, wp, wn = wu_r[...], wp_r[...], wn_r[...]

  eps = jnp.float32(1e-12)
  nu = jnp.maximum(jnp.sqrt(tuu), eps)
  np_ = jnp.maximum(jnp.sqrt(tpp), eps)
  nn_ = jnp.maximum(jnp.sqrt(tnn), eps)

  pos = up + wu * wp * tup / (nu * np_)
  neg = un + wu * wn * tun / (nu * nn_)
  l2 = (uu + pp + nn
        + (wu * wu) * tuu / (nu * nu)
        + (wp * wp) * tpp / (np_ * np_)
        + (wn * wn) * tnn / (nn_ * nn_))

  d = pos - neg
  base = jnp.maximum(-d, 0.0) + jnp.log1p(jnp.exp(-jnp.abs(d)))
  loss = jnp.mean(base) + _REG * (jnp.sum(l2) / (2.0 * _B))
  o_ref[0, 0] = loss


def kernel(u, pos_i, neg_i, ua_embed, ea_embed, u_text, i_text,
           w_utext, w_ctext):
  ut2 = _pack_halves(u_text)
  it2 = _pack_halves(i_text)
  wu1 = _squeeze_w(w_utext)
  wc1 = _squeeze_w(w_ctext)
  kstats = _kgat_stats(u, pos_i, neg_i, ua_embed, ea_embed)
  tstats = _text_stats(u, pos_i, neg_i, ut2, it2, wu1, wc1)
  loss = pl.pallas_call(
      _ep_body,
      out_shape=jax.ShapeDtypeStruct((1, 1), jnp.float32),
      out_specs=pl.BlockSpec(memory_space=pltpu.SMEM),
  )(*(kstats + tstats))
  return loss[0, 0]


# speedup vs baseline: 1.6672x; 1.6672x over previous
import functools

import jax
import jax.numpy as jnp
from jax import lax
from jax.experimental import pallas as pl
from jax.experimental.pallas import tpu as pltpu
from jax.experimental.pallas import tpu_sc as plsc

_B = 16384
_KD = 128
_TD = 64
_NC = 2
_NS = 16
_NW = _NC * _NS
_PER_W = _B // _NW
_C = 64
_ROUNDS = _PER_W // _C
_GROUPS = _C // 16
_REG = 1e-05
_HU = 50000
_HI = 25000

_MESH = plsc.VectorSubcoreMesh(
    core_axis_name="c", subcore_axis_name="s",
    num_cores=_NC, num_subcores=_NS)
_PARAMS = pltpu.CompilerParams(needs_layout_passes=False)


def _kgat_body(u_ref, pos_ref, neg_ref, ua_ref, ea_ref,
               o0, o1, o2, o3, o4,
               idx_u, idx_p, idx_n,
               ru0, rp0, rn0, ru1, rp1, rn1, staging, sem0, sem1):
  outs = (o0, o1, o2, o3, o4)
  bufs = ((ru0, rp0, rn0), (ru1, rp1, rn1))
  sems = (sem0, sem1)
  wid = lax.axis_index("s") * _NC + lax.axis_index("c")
  base = wid * _PER_W
  lane = lax.iota(jnp.int32, 16)

  pltpu.sync_copy(u_ref.at[pl.ds(base, _PER_W)], idx_u)
  pltpu.sync_copy(pos_ref.at[pl.ds(base, _PER_W)], idx_p)
  pltpu.sync_copy(neg_ref.at[pl.ds(base, _PER_W)], idx_n)

  def issue(r):
    p = r % 2
    sl = pl.ds(r * _C, _C)
    return [
        pltpu.async_copy(ua_ref.at[idx_u.at[sl]], bufs[p][0], sems[p]),
        pltpu.async_copy(ea_ref.at[idx_p.at[sl]], bufs[p][1], sems[p]),
        pltpu.async_copy(ea_ref.at[idx_n.at[sl]], bufs[p][2], sems[p]),
    ]

  descr = [issue(0), None]
  for r in range(_ROUNDS):
    p = r % 2
    if r + 1 < _ROUNDS:
      descr[1 - p] = issue(r + 1)
    for cp in descr[p]:
      cp.wait()
    ru, rp, rn = bufs[p]

    for g in range(_GROUPS):
      rows = lane + (g * 16)
      z = jnp.zeros((16,), jnp.float32)

      def kbody(d, accs, rows=rows, ru=ru, rp=rp, rn=rn):
        aup, aun, auu, app, ann = accs
        col = lax.bitwise_and(lane + d, _KD - 1)
        uv = plsc.load_gather(ru, [rows, col])
        pv = plsc.load_gather(rp, [rows, col])
        nv = plsc.load_gather(rn, [rows, col])
        return (aup + uv * pv, aun + uv * nv,
                auu + uv * uv, app + pv * pv, ann + nv * nv)

      accs = lax.fori_loop(0, _KD, kbody, (z, z, z, z, z), unroll=4)

      o = r * _C + g * 16
      for j, val in enumerate(accs):
        staging[pl.ds(j * _PER_W + o, 16)] = val

  for j in range(5):
    pltpu.sync_copy(staging.at[pl.ds(j * _PER_W, _PER_W)],
                    outs[j].at[pl.ds(base, _PER_W)])


_kgat_stats = functools.partial(
    pl.kernel,
    out_type=tuple(jax.ShapeDtypeStruct((_B,), jnp.float32)
                   for _ in range(5)),
    mesh=_MESH,
    scratch_types=[
        pltpu.VMEM((_PER_W,), jnp.int32),
        pltpu.VMEM((_PER_W,), jnp.int32),
        pltpu.VMEM((_PER_W,), jnp.int32),
        pltpu.VMEM((_C, _KD), jnp.float32),
        pltpu.VMEM((_C, _KD), jnp.float32),
        pltpu.VMEM((_C, _KD), jnp.float32),
        pltpu.VMEM((_C, _KD), jnp.float32),
        pltpu.VMEM((_C, _KD), jnp.float32),
        pltpu.VMEM((_C, _KD), jnp.float32),
        pltpu.VMEM((5 * _PER_W,), jnp.float32),
        pltpu.SemaphoreType.DMA,
        pltpu.SemaphoreType.DMA,
    ],
    compiler_params=_PARAMS,
)(_kgat_body)


def _text_body(u_ref, pos_ref, neg_ref, ut_ref, it_ref, wu_ref, wc_ref,
               o0, o1, o2, o3, o4, o5, o6, o7,
               idx_u, idx_p, idx_n, tix_u, tix_p, tix_n,
               tu0, tp0, tn0, tu1, tp1, tn1,
               wus0, wps0, wns0, wus1, wps1, wns1, staging, sem0, sem1):
  outs = (o0, o1, o2, o3, o4, o5, o6, o7)
  bufs = ((tu0, tp0, tn0, wus0, wps0, wns0),
          (tu1, tp1, tn1, wus1, wps1, wns1))
  sems = (sem0, sem1)
  wid = lax.axis_index("s") * _NC + lax.axis_index("c")
  base = wid * _PER_W
  lane = lax.iota(jnp.int32, 16)

  pltpu.sync_copy(u_ref.at[pl.ds(base, _PER_W)], idx_u)
  pltpu.sync_copy(pos_ref.at[pl.ds(base, _PER_W)], idx_p)
  pltpu.sync_copy(neg_ref.at[pl.ds(base, _PER_W)], idx_n)
  for g in range(_PER_W // 16):
    sl = pl.ds(g * 16, 16)
    vu, vp, vn = idx_u[sl], idx_p[sl], idx_n[sl]
    tix_u[sl] = jnp.where(vu < _HU, vu, vu - _HU)
    tix_p[sl] = jnp.where(vp < _HI, vp, vp - _HI)
    tix_n[sl] = jnp.where(vn < _HI, vn, vn - _HI)

  def issue(r):
    p = r % 2
    sl = pl.ds(r * _C, _C)
    b = bufs[p]
    return [
        pltpu.async_copy(ut_ref.at[tix_u.at[sl]], b[0], sems[p]),
        pltpu.async_copy(it_ref.at[tix_p.at[sl]], b[1], sems[p]),
        pltpu.async_copy(it_ref.at[tix_n.at[sl]], b[2], sems[p]),
        pltpu.async_copy(wu_ref.at[idx_u.at[sl]], b[3], sems[p]),
        pltpu.async_copy(wc_ref.at[idx_p.at[sl]], b[4], sems[p]),
        pltpu.async_copy(wc_ref.at[idx_n.at[sl]], b[5], sems[p]),
    ]

  descr = [issue(0), None]
  for r in range(_ROUNDS):
    p = r % 2
    if r + 1 < _ROUNDS:
      descr[1 - p] = issue(r + 1)
    for cp in descr[p]:
      cp.wait()
    tu, tp, tn, wus, wps, wns = bufs[p]

    for g in range(_GROUPS):
      rows = lane + (g * 16)
      z = jnp.zeros((16,), jnp.float32)
      sl16 = pl.ds(r * _C + g * 16, 16)

      half_u = jnp.where(idx_u[sl16] < _HU, 0, _TD)
      half_p = jnp.where(idx_p[sl16] < _HI, 0, _TD)
      half_n = jnp.where(idx_n[sl16] < _HI, 0, _TD)

      def tbody(d, accs, rows=rows, hu=half_u, hp=half_p, hn=half_n,
                tu=tu, tp=tp, tn=tn):
        bup, bun, buu, bpp, bnn = accs
        rot = lax.bitwise_and(lane + d, _TD - 1)
        uv = plsc.load_gather(tu, [rows, hu + rot])
        pv = plsc.load_gather(tp, [rows, hp + rot])
        nv = plsc.load_gather(tn, [rows, hn + rot])
        return (bup + uv * pv, bun + uv * nv,
                buu + uv * uv, bpp + pv * pv, bnn + nv * nv)

      accs = lax.fori_loop(0, _TD, tbody, (z, z, z, z, z), unroll=4)

      gsl = pl.ds(g * 16, 16)
      wuv = wus[gsl]
      wpv = wps[gsl]
      wnv = wns[gsl]

      o = r * _C + g * 16
      for j, val in enumerate(accs + (wuv, wpv, wnv)):
        staging[pl.ds(j * _PER_W + o, 16)] = val

  for j in range(8):
    pltpu.sync_copy(staging.at[pl.ds(j * _PER_W, _PER_W)],
                    outs[j].at[pl.ds(base, _PER_W)])


_text_stats = functools.partial(
    pl.kernel,
    out_type=tuple(jax.ShapeDtypeStruct((_B,), jnp.float32)
                   for _ in range(8)),
    mesh=_MESH,
    scratch_types=[
        pltpu.VMEM((_PER_W,), jnp.int32),
        pltpu.VMEM((_PER_W,), jnp.int32),
        pltpu.VMEM((_PER_W,), jnp.int32),
        pltpu.VMEM((_PER_W,), jnp.int32),
        pltpu.VMEM((_PER_W,), jnp.int32),
        pltpu.VMEM((_PER_W,), jnp.int32),
        pltpu.VMEM((_C, _KD), jnp.float32),
        pltpu.VMEM((_C, _KD), jnp.float32),
        pltpu.VMEM((_C, _KD), jnp.float32),
        pltpu.VMEM((_C, _KD), jnp.float32),
        pltpu.VMEM((_C, _KD), jnp.float32),
        pltpu.VMEM((_C, _KD), jnp.float32),
        pltpu.VMEM((_C,), jnp.float32),
        pltpu.VMEM((_C,), jnp.float32),
        pltpu.VMEM((_C,), jnp.float32),
        pltpu.VMEM((_C,), jnp.float32),
        pltpu.VMEM((_C,), jnp.float32),
        pltpu.VMEM((_C,), jnp.float32),
        pltpu.VMEM((8 * _PER_W,), jnp.float32),
        pltpu.SemaphoreType.DMA,
        pltpu.SemaphoreType.DMA,
    ],
    compiler_params=_PARAMS,
)(_text_body)


def _keep_on_tc(x):
  return jnp.where(jnp.isnan(x), jnp.float32(0), x)


def _ep_body(up_r, un_r, uu_r, pp_r, nn_r, tup_r, tun_r, tuu_r, tpp_r,
             tnn_r, wu_r, wp_r, wn_r, o_ref):
  up, un, uu, pp, nn = up_r[...], un_r[...], uu_r[...], pp_r[...], nn_r[...]
  tup, tun, tuu, tpp, tnn = (tup_r[...], tun_r[...], tuu_r[...],
                             tpp_r[...], tnn_r[...])
  wu, wp, wn = wu_r[...], wp_r[...], wn_r[...]

  eps = jnp.float32(1e-12)
  nu = jnp.maximum(jnp.sqrt(tuu), eps)
  np_ = jnp.maximum(jnp.sqrt(tpp), eps)
  nn_ = jnp.maximum(jnp.sqrt(tnn), eps)

  pos = up + wu * wp * tup / (nu * np_)
  neg = un + wu * wn * tun / (nu * nn_)
  l2 = (uu + pp + nn
        + (wu * wu) * tuu / (nu * nu)
        + (wp * wp) * tpp / (np_ * np_)
        + (wn * wn) * tnn / (nn_ * nn_))

  d = pos - neg
  base = jnp.maximum(-d, 0.0) + jnp.log1p(jnp.exp(-jnp.abs(d)))
  loss = jnp.mean(base) + _REG * (jnp.sum(l2) / (2.0 * _B))
  o_ref[0, 0] = loss


def kernel(u, pos_i, neg_i, ua_embed, ea_embed, u_text, i_text,
           w_utext, w_ctext):
  nu_ = u_text.shape[0]
  ni_ = i_text.shape[0]
  ut2 = _keep_on_tc(jnp.concatenate(
      [u_text[:nu_ // 2], u_text[nu_ // 2:]], axis=1))
  it2 = _keep_on_tc(jnp.concatenate(
      [i_text[:ni_ // 2], i_text[ni_ // 2:]], axis=1))
  wu1 = _keep_on_tc(w_utext[:, 0])
  wc1 = _keep_on_tc(w_ctext[:, 0])
  kstats = _kgat_stats(u, pos_i, neg_i, ua_embed, ea_embed)
  tstats = _text_stats(u, pos_i, neg_i, ut2, it2, wu1, wc1)
  loss = pl.pallas_call(
      _ep_body,
      out_shape=jax.ShapeDtypeStruct((1, 1), jnp.float32),
      out_specs=pl.BlockSpec(memory_space=pltpu.SMEM),
  )(*(kstats + tstats))
  return loss[0, 0]


# speedup vs baseline: 2.4914x; 1.4943x over previous
import functools

import jax
import jax.numpy as jnp
from jax import lax
from jax.experimental import pallas as pl
from jax.experimental.pallas import tpu as pltpu
from jax.experimental.pallas import tpu_sc as plsc

_B = 16384
_KD = 128
_TD = 64
_NC = 2
_NS = 16
_NW = _NC * _NS
_PER_W = _B // _NW
_C = 64
_ROUNDS = _PER_W // _C
_GROUPS = _C // 16
_REG = 1e-05
_HU = 50000
_HI = 25000

_MESH = plsc.VectorSubcoreMesh(
    core_axis_name="c", subcore_axis_name="s",
    num_cores=_NC, num_subcores=_NS)
_PARAMS = pltpu.CompilerParams(needs_layout_passes=False)


def _kgat_body(u_ref, pos_ref, neg_ref, ua_ref, ea_ref,
               o0, o1, o2, o3, o4,
               idx_u, idx_p, idx_n,
               ru0, rp0, rn0, ru1, rp1, rn1, staging, sem0, sem1):
  outs = (o0, o1, o2, o3, o4)
  bufs = ((ru0, rp0, rn0), (ru1, rp1, rn1))
  sems = (sem0, sem1)
  wid = lax.axis_index("s") * _NC + lax.axis_index("c")
  base = wid * _PER_W
  lane = lax.iota(jnp.int32, 16)

  pltpu.sync_copy(u_ref.at[pl.ds(base, _PER_W)], idx_u)
  pltpu.sync_copy(pos_ref.at[pl.ds(base, _PER_W)], idx_p)
  pltpu.sync_copy(neg_ref.at[pl.ds(base, _PER_W)], idx_n)

  def issue(r):
    p = r % 2
    sl = pl.ds(r * _C, _C)
    return [
        pltpu.async_copy(ua_ref.at[idx_u.at[sl]], bufs[p][0], sems[p]),
        pltpu.async_copy(ea_ref.at[idx_p.at[sl]], bufs[p][1], sems[p]),
        pltpu.async_copy(ea_ref.at[idx_n.at[sl]], bufs[p][2], sems[p]),
    ]

  descr = [issue(0), None]
  for r in range(_ROUNDS):
    p = r % 2
    if r + 1 < _ROUNDS:
      descr[1 - p] = issue(r + 1)
    for cp in descr[p]:
      cp.wait()
    ru, rp, rn = bufs[p]

    for g in range(_GROUPS):
      rows = lane + (g * 16)
      z = jnp.zeros((16,), jnp.float32)

      def kbody(d, accs, rows=rows, ru=ru, rp=rp, rn=rn):
        aup, aun, auu, app, ann = accs
        col = lax.bitwise_and(lane + d, _KD - 1)
        uv = plsc.load_gather(ru, [rows, col])
        pv = plsc.load_gather(rp, [rows, col])
        nv = plsc.load_gather(rn, [rows, col])
        return (aup + uv * pv, aun + uv * nv,
                auu + uv * uv, app + pv * pv, ann + nv * nv)

      accs = lax.fori_loop(0, _KD, kbody, (z, z, z, z, z), unroll=4)

      o = r * _C + g * 16
      for j, val in enumerate(accs):
        staging[pl.ds(j * _PER_W + o, 16)] = val

  for j in range(5):
    pltpu.sync_copy(staging.at[pl.ds(j * _PER_W, _PER_W)],
                    outs[j].at[pl.ds(base, _PER_W)])


_kgat_stats = functools.partial(
    pl.kernel,
    out_type=tuple(jax.ShapeDtypeStruct((_B,), jnp.float32)
                   for _ in range(5)),
    mesh=_MESH,
    scratch_types=[
        pltpu.VMEM((_PER_W,), jnp.int32),
        pltpu.VMEM((_PER_W,), jnp.int32),
        pltpu.VMEM((_PER_W,), jnp.int32),
        pltpu.VMEM((_C, _KD), jnp.float32),
        pltpu.VMEM((_C, _KD), jnp.float32),
        pltpu.VMEM((_C, _KD), jnp.float32),
        pltpu.VMEM((_C, _KD), jnp.float32),
        pltpu.VMEM((_C, _KD), jnp.float32),
        pltpu.VMEM((_C, _KD), jnp.float32),
        pltpu.VMEM((5 * _PER_W,), jnp.float32),
        pltpu.SemaphoreType.DMA,
        pltpu.SemaphoreType.DMA,
    ],
    compiler_params=_PARAMS,
)(_kgat_body)


def _text_body(u_ref, pos_ref, neg_ref, ut_ref, it_ref, wu_ref, wc_ref,
               o0, o1, o2, o3, o4, o5, o6, o7,
               idx_u, idx_p, idx_n, tix_u, tix_p, tix_n,
               tu0, tp0, tn0, tu1, tp1, tn1,
               wus0, wps0, wns0, wus1, wps1, wns1, staging, sem0, sem1):
  outs = (o0, o1, o2, o3, o4, o5, o6, o7)
  bufs = ((tu0, tp0, tn0, wus0, wps0, wns0),
          (tu1, tp1, tn1, wus1, wps1, wns1))
  sems = (sem0, sem1)
  wid = lax.axis_index("s") * _NC + lax.axis_index("c")
  base = wid * _PER_W
  lane = lax.iota(jnp.int32, 16)

  pltpu.sync_copy(u_ref.at[pl.ds(base, _PER_W)], idx_u)
  pltpu.sync_copy(pos_ref.at[pl.ds(base, _PER_W)], idx_p)
  pltpu.sync_copy(neg_ref.at[pl.ds(base, _PER_W)], idx_n)
  for g in range(_PER_W // 16):
    sl = pl.ds(g * 16, 16)
    tix_u[sl] = lax.shift_right_logical(idx_u[sl], 1)
    tix_p[sl] = lax.shift_right_logical(idx_p[sl], 1)
    tix_n[sl] = lax.shift_right_logical(idx_n[sl], 1)

  def issue(r):
    p = r % 2
    sl = pl.ds(r * _C, _C)
    b = bufs[p]
    return [
        pltpu.async_copy(ut_ref.at[tix_u.at[sl]], b[0], sems[p]),
        pltpu.async_copy(it_ref.at[tix_p.at[sl]], b[1], sems[p]),
        pltpu.async_copy(it_ref.at[tix_n.at[sl]], b[2], sems[p]),
        pltpu.async_copy(wu_ref.at[idx_u.at[sl]], b[3], sems[p]),
        pltpu.async_copy(wc_ref.at[idx_p.at[sl]], b[4], sems[p]),
        pltpu.async_copy(wc_ref.at[idx_n.at[sl]], b[5], sems[p]),
    ]

  descr = [issue(0), None]
  for r in range(_ROUNDS):
    p = r % 2
    if r + 1 < _ROUNDS:
      descr[1 - p] = issue(r + 1)
    for cp in descr[p]:
      cp.wait()
    tu, tp, tn, wus, wps, wns = bufs[p]

    for g in range(_GROUPS):
      rows = lane + (g * 16)
      z = jnp.zeros((16,), jnp.float32)
      sl16 = pl.ds(r * _C + g * 16, 16)

      half_u = lax.bitwise_and(idx_u[sl16], 1) * _TD
      half_p = lax.bitwise_and(idx_p[sl16], 1) * _TD
      half_n = lax.bitwise_and(idx_n[sl16], 1) * _TD

      def tbody(d, accs, rows=rows, hu=half_u, hp=half_p, hn=half_n,
                tu=tu, tp=tp, tn=tn):
        bup, bun, buu, bpp, bnn = accs
        rot = lax.bitwise_and(lane + d, _TD - 1)
        uv = plsc.load_gather(tu, [rows, hu + rot])
        pv = plsc.load_gather(tp, [rows, hp + rot])
        nv = plsc.load_gather(tn, [rows, hn + rot])
        return (bup + uv * pv, bun + uv * nv,
                buu + uv * uv, bpp + pv * pv, bnn + nv * nv)

      accs = lax.fori_loop(0, _TD, tbody, (z, z, z, z, z), unroll=4)

      gsl = pl.ds(g * 16, 16)
      wuv = wus[gsl]
      wpv = wps[gsl]
      wnv = wns[gsl]

      o = r * _C + g * 16
      for j, val in enumerate(accs + (wuv, wpv, wnv)):
        staging[pl.ds(j * _PER_W + o, 16)] = val

  for j in range(8):
    pltpu.sync_copy(staging.at[pl.ds(j * _PER_W, _PER_W)],
                    outs[j].at[pl.ds(base, _PER_W)])


_text_stats = functools.partial(
    pl.kernel,
    out_type=tuple(jax.ShapeDtypeStruct((_B,), jnp.float32)
                   for _ in range(8)),
    mesh=_MESH,
    scratch_types=[
        pltpu.VMEM((_PER_W,), jnp.int32),
        pltpu.VMEM((_PER_W,), jnp.int32),
        pltpu.VMEM((_PER_W,), jnp.int32),
        pltpu.VMEM((_PER_W,), jnp.int32),
        pltpu.VMEM((_PER_W,), jnp.int32),
        pltpu.VMEM((_PER_W,), jnp.int32),
        pltpu.VMEM((_C, _KD), jnp.float32),
        pltpu.VMEM((_C, _KD), jnp.float32),
        pltpu.VMEM((_C, _KD), jnp.float32),
        pltpu.VMEM((_C, _KD), jnp.float32),
        pltpu.VMEM((_C, _KD), jnp.float32),
        pltpu.VMEM((_C, _KD), jnp.float32),
        pltpu.VMEM((_C,), jnp.float32),
        pltpu.VMEM((_C,), jnp.float32),
        pltpu.VMEM((_C,), jnp.float32),
        pltpu.VMEM((_C,), jnp.float32),
        pltpu.VMEM((_C,), jnp.float32),
        pltpu.VMEM((_C,), jnp.float32),
        pltpu.VMEM((8 * _PER_W,), jnp.float32),
        pltpu.SemaphoreType.DMA,
        pltpu.SemaphoreType.DMA,
    ],
    compiler_params=_PARAMS,
)(_text_body)


def _keep_on_tc(x):
  return jnp.where(jnp.isnan(x), jnp.float32(0), x)


def _ep_body(up_r, un_r, uu_r, pp_r, nn_r, tup_r, tun_r, tuu_r, tpp_r,
             tnn_r, wu_r, wp_r, wn_r, o_ref):
  up, un, uu, pp, nn = up_r[...], un_r[...], uu_r[...], pp_r[...], nn_r[...]
  tup, tun, tuu, tpp, tnn = (tup_r[...], tun_r[...], tuu_r[...],
                             tpp_r[...], tnn_r[...])
  wu, wp, wn = wu_r[...], wp_r[...], wn_r[...]

  eps = jnp.float32(1e-12)
  nu = jnp.maximum(jnp.sqrt(tuu), eps)
  np_ = jnp.maximum(jnp.sqrt(tpp), eps)
  nn_ = jnp.maximum(jnp.sqrt(tnn), eps)

  pos = up + wu * wp * tup / (nu * np_)
  neg = un + wu * wn * tun / (nu * nn_)
  l2 = (uu + pp + nn
        + (wu * wu) * tuu / (nu * nu)
        + (wp * wp) * tpp / (np_ * np_)
        + (wn * wn) * tnn / (nn_ * nn_))

  d = pos - neg
  base = jnp.maximum(-d, 0.0) + jnp.log1p(jnp.exp(-jnp.abs(d)))
  loss = jnp.mean(base) + _REG * (jnp.sum(l2) / (2.0 * _B))
  o_ref[0, 0] = loss


def kernel(u, pos_i, neg_i, ua_embed, ea_embed, u_text, i_text,
           w_utext, w_ctext):
  kstats = _kgat_stats(u, pos_i, neg_i, ua_embed, ea_embed)
  ut2 = jnp.reshape(u_text, (u_text.shape[0] // 2, 2 * _TD))
  it2 = jnp.reshape(i_text, (i_text.shape[0] // 2, 2 * _TD))
  wu1 = _keep_on_tc(w_utext[:, 0])
  wc1 = _keep_on_tc(w_ctext[:, 0])
  tstats = _text_stats(u, pos_i, neg_i, ut2, it2, wu1, wc1)
  loss = pl.pallas_call(
      _ep_body,
      out_shape=jax.ShapeDtypeStruct((1, 1), jnp.float32),
      out_specs=pl.BlockSpec(memory_space=pltpu.SMEM),
  )(*(kstats + tstats))
  return loss[0, 0]


# speedup vs baseline: 2.5010x; 1.0039x over previous
import functools

import jax
import jax.numpy as jnp
from jax import lax
from jax.experimental import pallas as pl
from jax.experimental.pallas import tpu as pltpu
from jax.experimental.pallas import tpu_sc as plsc

_B = 16384
_KD = 128
_TD = 64
_NC = 2
_NS = 16
_NW = _NC * _NS
_PER_W = _B // _NW
_C = 128
_ROUNDS = _PER_W // _C
_GROUPS = _C // 16
_REG = 1e-05
_HU = 50000
_HI = 25000

_MESH = plsc.VectorSubcoreMesh(
    core_axis_name="c", subcore_axis_name="s",
    num_cores=_NC, num_subcores=_NS)
_PARAMS = pltpu.CompilerParams(needs_layout_passes=False)


def _kgat_body(u_ref, pos_ref, neg_ref, ua_ref, ea_ref,
               o0, o1, o2, o3, o4,
               idx_u, idx_p, idx_n,
               ru0, rp0, rn0, ru1, rp1, rn1, staging, sem0, sem1):
  outs = (o0, o1, o2, o3, o4)
  bufs = ((ru0, rp0, rn0), (ru1, rp1, rn1))
  sems = (sem0, sem1)
  wid = lax.axis_index("s") * _NC + lax.axis_index("c")
  base = wid * _PER_W
  lane = lax.iota(jnp.int32, 16)

  pltpu.sync_copy(u_ref.at[pl.ds(base, _PER_W)], idx_u)
  pltpu.sync_copy(pos_ref.at[pl.ds(base, _PER_W)], idx_p)
  pltpu.sync_copy(neg_ref.at[pl.ds(base, _PER_W)], idx_n)

  def issue(r):
    p = r % 2
    sl = pl.ds(r * _C, _C)
    return [
        pltpu.async_copy(ua_ref.at[idx_u.at[sl]], bufs[p][0], sems[p]),
        pltpu.async_copy(ea_ref.at[idx_p.at[sl]], bufs[p][1], sems[p]),
        pltpu.async_copy(ea_ref.at[idx_n.at[sl]], bufs[p][2], sems[p]),
    ]

  descr = [issue(0), None]
  for r in range(_ROUNDS):
    p = r % 2
    if r + 1 < _ROUNDS:
      descr[1 - p] = issue(r + 1)
    for cp in descr[p]:
      cp.wait()
    ru, rp, rn = bufs[p]

    for g in range(_GROUPS):
      rows = lane + (g * 16)
      z = jnp.zeros((16,), jnp.float32)

      def kbody(d, accs, rows=rows, ru=ru, rp=rp, rn=rn):
        aup, aun, auu, app, ann = accs
        col = lax.bitwise_and(lane + d, _KD - 1)
        uv = plsc.load_gather(ru, [rows, col])
        pv = plsc.load_gather(rp, [rows, col])
        nv = plsc.load_gather(rn, [rows, col])
        return (aup + uv * pv, aun + uv * nv,
                auu + uv * uv, app + pv * pv, ann + nv * nv)

      accs = lax.fori_loop(0, _KD, kbody, (z, z, z, z, z), unroll=4)

      o = r * _C + g * 16
      for j, val in enumerate(accs):
        staging[pl.ds(j * _PER_W + o, 16)] = val

  for j in range(5):
    pltpu.sync_copy(staging.at[pl.ds(j * _PER_W, _PER_W)],
                    outs[j].at[pl.ds(base, _PER_W)])


_kgat_stats = functools.partial(
    pl.kernel,
    out_type=tuple(jax.ShapeDtypeStruct((_B,), jnp.float32)
                   for _ in range(5)),
    mesh=_MESH,
    scratch_types=[
        pltpu.VMEM((_PER_W,), jnp.int32),
        pltpu.VMEM((_PER_W,), jnp.int32),
        pltpu.VMEM((_PER_W,), jnp.int32),
        pltpu.VMEM((_C, _KD), jnp.float32),
        pltpu.VMEM((_C, _KD), jnp.float32),
        pltpu.VMEM((_C, _KD), jnp.float32),
        pltpu.VMEM((_C, _KD), jnp.float32),
        pltpu.VMEM((_C, _KD), jnp.float32),
        pltpu.VMEM((_C, _KD), jnp.float32),
        pltpu.VMEM((5 * _PER_W,), jnp.float32),
        pltpu.SemaphoreType.DMA,
        pltpu.SemaphoreType.DMA,
    ],
    compiler_params=_PARAMS,
)(_kgat_body)


def _text_body(u_ref, pos_ref, neg_ref, ut_ref, it_ref, wu_ref, wc_ref,
               o0, o1, o2, o3, o4, o5, o6, o7,
               idx_u, idx_p, idx_n, tix_u, tix_p, tix_n,
               tu0, tp0, tn0, tu1, tp1, tn1,
               wus0, wps0, wns0, wus1, wps1, wns1, staging, sem0, sem1):
  outs = (o0, o1, o2, o3, o4, o5, o6, o7)
  bufs = ((tu0, tp0, tn0, wus0, wps0, wns0),
          (tu1, tp1, tn1, wus1, wps1, wns1))
  sems = (sem0, sem1)
  wid = lax.axis_index("s") * _NC + lax.axis_index("c")
  base = wid * _PER_W
  lane = lax.iota(jnp.int32, 16)

  pltpu.sync_copy(u_ref.at[pl.ds(base, _PER_W)], idx_u)
  pltpu.sync_copy(pos_ref.at[pl.ds(base, _PER_W)], idx_p)
  pltpu.sync_copy(neg_ref.at[pl.ds(base, _PER_W)], idx_n)
  for g in range(_PER_W // 16):
    sl = pl.ds(g * 16, 16)
    tix_u[sl] = lax.shift_right_logical(idx_u[sl], 1)
    tix_p[sl] = lax.shift_right_logical(idx_p[sl], 1)
    tix_n[sl] = lax.shift_right_logical(idx_n[sl], 1)

  def issue(r):
    p = r % 2
    sl = pl.ds(r * _C, _C)
    b = bufs[p]
    return [
        pltpu.async_copy(ut_ref.at[tix_u.at[sl]], b[0], sems[p]),
        pltpu.async_copy(it_ref.at[tix_p.at[sl]], b[1], sems[p]),
        pltpu.async_copy(it_ref.at[tix_n.at[sl]], b[2], sems[p]),
        pltpu.async_copy(wu_ref.at[idx_u.at[sl]], b[3], sems[p]),
        pltpu.async_copy(wc_ref.at[idx_p.at[sl]], b[4], sems[p]),
        pltpu.async_copy(wc_ref.at[idx_n.at[sl]], b[5], sems[p]),
    ]

  descr = [issue(0), None]
  for r in range(_ROUNDS):
    p = r % 2
    if r + 1 < _ROUNDS:
      descr[1 - p] = issue(r + 1)
    for cp in descr[p]:
      cp.wait()
    tu, tp, tn, wus, wps, wns = bufs[p]

    for g in range(_GROUPS):
      rows = lane + (g * 16)
      z = jnp.zeros((16,), jnp.float32)
      sl16 = pl.ds(r * _C + g * 16, 16)

      half_u = lax.bitwise_and(idx_u[sl16], 1) * _TD
      half_p = lax.bitwise_and(idx_p[sl16], 1) * _TD
      half_n = lax.bitwise_and(idx_n[sl16], 1) * _TD

      def tbody(d, accs, rows=rows, hu=half_u, hp=half_p, hn=half_n,
                tu=tu, tp=tp, tn=tn):
        bup, bun, buu, bpp, bnn = accs
        rot = lax.bitwise_and(lane + d, _TD - 1)
        uv = plsc.load_gather(tu, [rows, hu + rot])
        pv = plsc.load_gather(tp, [rows, hp + rot])
        nv = plsc.load_gather(tn, [rows, hn + rot])
        return (bup + uv * pv, bun + uv * nv,
                buu + uv * uv, bpp + pv * pv, bnn + nv * nv)

      accs = lax.fori_loop(0, _TD, tbody, (z, z, z, z, z), unroll=4)

      gsl = pl.ds(g * 16, 16)
      wuv = wus[gsl]
      wpv = wps[gsl]
      wnv = wns[gsl]

      o = r * _C + g * 16
      for j, val in enumerate(accs + (wuv, wpv, wnv)):
        staging[pl.ds(j * _PER_W + o, 16)] = val

  for j in range(8):
    pltpu.sync_copy(staging.at[pl.ds(j * _PER_W, _PER_W)],
                    outs[j].at[pl.ds(base, _PER_W)])


_text_stats = functools.partial(
    pl.kernel,
    out_type=tuple(jax.ShapeDtypeStruct((_B,), jnp.float32)
                   for _ in range(8)),
    mesh=_MESH,
    scratch_types=[
        pltpu.VMEM((_PER_W,), jnp.int32),
        pltpu.VMEM((_PER_W,), jnp.int32),
        pltpu.VMEM((_PER_W,), jnp.int32),
        pltpu.VMEM((_PER_W,), jnp.int32),
        pltpu.VMEM((_PER_W,), jnp.int32),
        pltpu.VMEM((_PER_W,), jnp.int32),
        pltpu.VMEM((_C, _KD), jnp.float32),
        pltpu.VMEM((_C, _KD), jnp.float32),
        pltpu.VMEM((_C, _KD), jnp.float32),
        pltpu.VMEM((_C, _KD), jnp.float32),
        pltpu.VMEM((_C, _KD), jnp.float32),
        pltpu.VMEM((_C, _KD), jnp.float32),
        pltpu.VMEM((_C,), jnp.float32),
        pltpu.VMEM((_C,), jnp.float32),
        pltpu.VMEM((_C,), jnp.float32),
        pltpu.VMEM((_C,), jnp.float32),
        pltpu.VMEM((_C,), jnp.float32),
        pltpu.VMEM((_C,), jnp.float32),
        pltpu.VMEM((8 * _PER_W,), jnp.float32),
        pltpu.SemaphoreType.DMA,
        pltpu.SemaphoreType.DMA,
    ],
    compiler_params=_PARAMS,
)(_text_body)


def _keep_on_tc(x):
  return jnp.where(jnp.isnan(x), jnp.float32(0), x)


def _ep_body(up_r, un_r, uu_r, pp_r, nn_r, tup_r, tun_r, tuu_r, tpp_r,
             tnn_r, wu_r, wp_r, wn_r, o_ref):
  up, un, uu, pp, nn = up_r[...], un_r[...], uu_r[...], pp_r[...], nn_r[...]
  tup, tun, tuu, tpp, tnn = (tup_r[...], tun_r[...], tuu_r[...],
                             tpp_r[...], tnn_r[...])
  wu, wp, wn = wu_r[...], wp_r[...], wn_r[...]

  eps = jnp.float32(1e-12)
  nu = jnp.maximum(jnp.sqrt(tuu), eps)
  np_ = jnp.maximum(jnp.sqrt(tpp), eps)
  nn_ = jnp.maximum(jnp.sqrt(tnn), eps)

  pos = up + wu * wp * tup / (nu * np_)
  neg = un + wu * wn * tun / (nu * nn_)
  l2 = (uu + pp + nn
        + (wu * wu) * tuu / (nu * nu)
        + (wp * wp) * tpp / (np_ * np_)
        + (wn * wn) * tnn / (nn_ * nn_))

  d = pos - neg
  base = jnp.maximum(-d, 0.0) + jnp.log1p(jnp.exp(-jnp.abs(d)))
  loss = jnp.mean(base) + _REG * (jnp.sum(l2) / (2.0 * _B))
  o_ref[0, 0] = loss


def kernel(u, pos_i, neg_i, ua_embed, ea_embed, u_text, i_text,
           w_utext, w_ctext):
  kstats = _kgat_stats(u, pos_i, neg_i, ua_embed, ea_embed)
  ut2 = jnp.reshape(u_text, (u_text.shape[0] // 2, 2 * _TD))
  it2 = jnp.reshape(i_text, (i_text.shape[0] // 2, 2 * _TD))
  wu1 = _keep_on_tc(w_utext[:, 0])
  wc1 = _keep_on_tc(w_ctext[:, 0])
  tstats = _text_stats(u, pos_i, neg_i, ut2, it2, wu1, wc1)
  loss = pl.pallas_call(
      _ep_body,
      out_shape=jax.ShapeDtypeStruct((1, 1), jnp.float32),
      out_specs=pl.BlockSpec(memory_space=pltpu.SMEM),
  )(*(kstats + tstats))
  return loss[0, 0]
